# Initial kernel scaffold; baseline (speedup 1.0000x reference)
import jax
import jax.numpy as jnp
from jax.experimental import pallas as pl


def kernel(z, edge_index, W1, a_src1, a_dst1, b1, W2, a_src2, a_dst2, b2, Wlin, blin):
    raise NotImplementedError("write your pallas kernel here")


# speedup vs baseline: 20.9250x; 20.9250x over previous
import functools

import jax
import jax.numpy as jnp
from jax import lax
from jax.experimental import pallas as pl
from jax.experimental.pallas import tpu as pltpu
from jax.experimental.pallas import tpu_sc as plsc

NN = 10000
NP = 12288
EE = 160000
H = 8
C = 64
F = H * C

NB = 256
GRID = NP // NB

NSUB = 16
CHUNK = 1536
NCH = 4
PADR = 64
EPT = EE // NSUB
BLK = 400
NBLK = EPT // BLK
SUB = 64
CSLOTS = 160
TROW = CSLOTS - 1


HP = 16


def _expand_attn(a):
    m = (jnp.eye(H, dtype=a.dtype)[:, None, :] * a[:, :, None]).reshape(F, H)
    return jnp.pad(m, ((0, 0), (0, HP - H)))


def _pre_body(x_ref, w_ref, eas_ref, ead_ref, xp_ref, als_ref, ald_ref, ees_ref):
    xp = jnp.dot(x_ref[...], w_ref[...], preferred_element_type=jnp.float32)
    al_s = jnp.dot(xp, eas_ref[...], preferred_element_type=jnp.float32)
    al_d = jnp.dot(xp, ead_ref[...], preferred_element_type=jnp.float32)
    s = al_s + al_d
    xp_ref[...] = xp
    als_ref[...] = al_s
    ald_ref[...] = al_d
    ees_ref[...] = jnp.exp(jnp.maximum(s, 0.2 * s))


def _tc_pre(x, w, eas, ead):
    d = x.shape[1]
    f32 = jnp.float32
    return pl.pallas_call(
        _pre_body,
        grid=(GRID,),
        in_specs=[
            pl.BlockSpec((NB, d), lambda i: (i, 0)),
            pl.BlockSpec((d, F), lambda i: (0, 0)),
            pl.BlockSpec((F, HP), lambda i: (0, 0)),
            pl.BlockSpec((F, HP), lambda i: (0, 0)),
        ],
        out_specs=[
            pl.BlockSpec((NB, F), lambda i: (i, 0)),
            pl.BlockSpec((NB, HP), lambda i: (i, 0)),
            pl.BlockSpec((NB, HP), lambda i: (i, 0)),
            pl.BlockSpec((NB, HP), lambda i: (i, 0)),
        ],
        out_shape=[
            jax.ShapeDtypeStruct((NP, F), f32),
            jax.ShapeDtypeStruct((NP, HP), f32),
            jax.ShapeDtypeStruct((NP, HP), f32),
            jax.ShapeDtypeStruct((NP, HP), f32),
        ],
    )(x, w, eas, ead)


def _combine(raw_ref, den_ref, ees_ref, xp_ref, b_ref, eexp_ref):
    ees = ees_ref[...][:, :H]
    invd = 1.0 / (den_ref[...][:, :H] + ees + 1e-16)
    ee_x = jnp.dot(ees, eexp_ref[...], preferred_element_type=jnp.float32)
    iv_x = jnp.dot(invd, eexp_ref[...], preferred_element_type=jnp.float32)
    return jnp.tanh((raw_ref[...] + ee_x * xp_ref[...]) * iv_x + b_ref[...])


def _mid_body(raw_ref, den_ref, ees_ref, xp_ref, b_ref, eexp_ref, w_ref,
              eas_ref, ead_ref, xp2_ref, als_ref, ald_ref, ees2_ref):
    x2 = _combine(raw_ref, den_ref, ees_ref, xp_ref, b_ref, eexp_ref)
    xp2 = jnp.dot(x2, w_ref[...], preferred_element_type=jnp.float32)
    al_s = jnp.dot(xp2, eas_ref[...], preferred_element_type=jnp.float32)
    al_d = jnp.dot(xp2, ead_ref[...], preferred_element_type=jnp.float32)
    s = al_s + al_d
    xp2_ref[...] = xp2
    als_ref[...] = al_s
    ald_ref[...] = al_d
    ees2_ref[...] = jnp.exp(jnp.maximum(s, 0.2 * s))


def _tc_mid(raw, den, ees, xp, b, eexp, w, eas, ead):
    f32 = jnp.float32
    return pl.pallas_call(
        _mid_body,
        grid=(GRID,),
        in_specs=[
            pl.BlockSpec((NB, F), lambda i: (i, 0)),
            pl.BlockSpec((NB, HP), lambda i: (i, 0)),
            pl.BlockSpec((NB, HP), lambda i: (i, 0)),
            pl.BlockSpec((NB, F), lambda i: (i, 0)),
            pl.BlockSpec((1, F), lambda i: (0, 0)),
            pl.BlockSpec((H, F), lambda i: (0, 0)),
            pl.BlockSpec((F, F), lambda i: (0, 0)),
            pl.BlockSpec((F, HP), lambda i: (0, 0)),
            pl.BlockSpec((F, HP), lambda i: (0, 0)),
        ],
        out_specs=[
            pl.BlockSpec((NB, F), lambda i: (i, 0)),
            pl.BlockSpec((NB, HP), lambda i: (i, 0)),
            pl.BlockSpec((NB, HP), lambda i: (i, 0)),
            pl.BlockSpec((NB, HP), lambda i: (i, 0)),
        ],
        out_shape=[
            jax.ShapeDtypeStruct((NP, F), f32),
            jax.ShapeDtypeStruct((NP, HP), f32),
            jax.ShapeDtypeStruct((NP, HP), f32),
            jax.ShapeDtypeStruct((NP, HP), f32),
        ],
    )(raw, den, ees, xp, b, eexp, w, eas, ead)


def _fin_body(raw_ref, den_ref, ees_ref, xp_ref, b_ref, eexp_ref, wl_ref,
              bl_ref, y_ref):
    x2 = _combine(raw_ref, den_ref, ees_ref, xp_ref, b_ref, eexp_ref)
    y_ref[...] = jnp.dot(x2, wl_ref[...],
                         preferred_element_type=jnp.float32) + bl_ref[...]


def _tc_fin(raw, den, ees, xp, b, eexp, wl, bl):
    return pl.pallas_call(
        _fin_body,
        grid=(GRID,),
        in_specs=[
            pl.BlockSpec((NB, F), lambda i: (i, 0)),
            pl.BlockSpec((NB, HP), lambda i: (i, 0)),
            pl.BlockSpec((NB, HP), lambda i: (i, 0)),
            pl.BlockSpec((NB, F), lambda i: (i, 0)),
            pl.BlockSpec((1, F), lambda i: (0, 0)),
            pl.BlockSpec((H, F), lambda i: (0, 0)),
            pl.BlockSpec((F, 128), lambda i: (0, 0)),
            pl.BlockSpec((1, 128), lambda i: (0, 0)),
        ],
        out_specs=pl.BlockSpec((NB, 128), lambda i: (i, 0)),
        out_shape=jax.ShapeDtypeStruct((NP, 128), jnp.float32),
    )(raw, den, ees, xp, b, eexp, wl, bl)


def _sc_msgpass(src, dst, al_s, al_d, xp):
    f32 = jnp.float32
    i32 = jnp.int32
    mesh = plsc.VectorSubcoreMesh(core_axis_name="c", subcore_axis_name="s")
    rpt = CHUNK // NSUB

    @functools.partial(
        pl.kernel,
        out_type=(jax.ShapeDtypeStruct((NP, F), f32),
                  jax.ShapeDtypeStruct((NP, HP), f32)),
        mesh=mesh,
        compiler_params=pltpu.CompilerParams(needs_layout_passes=False,
                                             use_tc_tiling_on_sc=False),
        scratch_types=(
            pltpu.VMEM((BLK,), i32),
            pltpu.VMEM((BLK,), i32),
            pltpu.VMEM((CSLOTS, SUB), i32),
            pltpu.VMEM((CSLOTS, SUB), i32),
            pltpu.VMEM((CSLOTS, SUB), i32),
            pltpu.VMEM((SUB, HP), f32),
            pltpu.VMEM((SUB, HP), f32),
            pltpu.VMEM((SUB, HP), f32),
            pltpu.VMEM((SUB, F), f32),
            pltpu.VMEM((16, F), f32),
            pltpu.VMEM((rpt, HP), f32),
            pltpu.VMEM_SHARED((CHUNK + PADR, F), f32),
            pltpu.VMEM_SHARED((CHUNK + PADR, HP), f32),
        ),
    )
    def k(src_h, dst_h, als_h, ald_h, xp_h, out_h, den_h,
          src_b, dst_b, src_c, dst_c, off_c,
          asg, adg, ee2, rows, zbuf, zden, out_sh, den_sh):
        cid = lax.axis_index("c")
        sid = lax.axis_index("s")
        lanes = lax.iota(i32, 16)
        zf = jnp.zeros((16,), f32)

        def zb_row(i, _):
            def zb_col(q, _):
                zbuf[i, pl.ds(q * 16, 16)] = zf
                return 0
            return lax.fori_loop(0, F // 16, zb_col, 0)
        lax.fori_loop(0, 16, zb_row, 0)

        def zd(i, _):
            zden[i, pl.ds(0, HP)] = zf
            return 0
        lax.fori_loop(0, rpt, zd, 0)

        r0 = sid * rpt
        e0 = sid * EPT

        for half in range(NCH):
            base = (cid * NCH + half) * CHUNK

            for t in range(rpt // 16):
                pltpu.sync_copy(zbuf, out_sh.at[pl.ds(r0 + t * 16, 16)])
            pltpu.sync_copy(zden, den_sh.at[pl.ds(r0, rpt)])
            plsc.subcore_barrier()

            def scan_blk(b, kk):
                pltpu.sync_copy(src_h.at[pl.ds(e0 + b * BLK, BLK)], src_b)
                pltpu.sync_copy(dst_h.at[pl.ds(e0 + b * BLK, BLK)], dst_b)

                def grp(g, kk):
                    sv = src_b[pl.ds(g * 16, 16)]
                    dv = dst_b[pl.ds(g * 16, 16)]
                    m = (dv >= base) & (dv < base + CHUNK)
                    ps = plsc.cumsum(m.astype(i32))
                    pos = kk + ps - 1
                    pr = jnp.where(m, pos >> 6, TROW)
                    pc = jnp.where(m, pos & (SUB - 1), lanes)
                    plsc.store_scatter(src_c, [pr, pc], sv)
                    plsc.store_scatter(dst_c, [pr, pc], dv)
                    plsc.store_scatter(off_c, [pr, pc], dv - base)
                    return kk + ps[15]
                return lax.fori_loop(0, BLK // 16, grp, kk)
            kk = lax.fori_loop(0, NBLK, scan_blk, jnp.asarray(0, i32))

            zi = jnp.zeros((16,), i32)
            for g in range(SUB // 16):
                pp = kk + g * 16 + lanes
                pr = pp >> 6
                pc = pp & (SUB - 1)
                plsc.store_scatter(src_c, [pr, pc], zi)
                plsc.store_scatter(dst_c, [pr, pc], zi)
                plsc.store_scatter(off_c, [pr, pc], CHUNK + g * 16 + lanes)
            nsub = (kk + (SUB - 1)) // SUB

            def proc(j, _):
                pltpu.sync_copy(als_h.at[src_c.at[j]], asg)
                pltpu.sync_copy(ald_h.at[dst_c.at[j]], adg)

                def eeg(e, _):
                    s = asg[e, pl.ds(0, HP)] + adg[e, pl.ds(0, HP)]
                    ee2[e, pl.ds(0, HP)] = jnp.exp(jnp.maximum(s, 0.2 * s))
                    return 0
                lax.fori_loop(0, SUB, eeg, 0)
                pltpu.sync_copy(ee2, den_sh.at[off_c.at[j]], add=True)
                pltpu.sync_copy(xp_h.at[src_c.at[j]], rows)

                def scale(e, _):
                    ev = ee2[e, pl.ds(0, HP)]
                    for h in range(H):
                        sv = jnp.full((16,), ev[h], f32)
                        for q in range(C // 16):
                            sl = pl.ds(h * C + q * 16, 16)
                            rows[e, sl] = rows[e, sl] * sv
                    return 0
                lax.fori_loop(0, SUB, scale, 0)
                pltpu.sync_copy(rows, out_sh.at[off_c.at[j]], add=True)
                return 0
            lax.fori_loop(0, nsub, proc, 0)
            plsc.subcore_barrier()

            pltpu.sync_copy(out_sh.at[pl.ds(r0, rpt)],
                            out_h.at[pl.ds(base + r0, rpt)])
            pltpu.sync_copy(den_sh.at[pl.ds(r0, rpt)],
                            den_h.at[pl.ds(base + r0, rpt)])
            plsc.subcore_barrier()

    return k(src, dst, al_s, al_d, xp)


def kernel(z, edge_index, W1, a_src1, a_dst1, b1, W2, a_src2, a_dst2, b2,
           Wlin, blin):
    f32 = jnp.float32
    src = edge_index[0].astype(jnp.int32)
    dst = edge_index[1].astype(jnp.int32)
    zp = jnp.pad(z, ((0, NP - NN), (0, 0)))

    eas1, ead1 = _expand_attn(a_src1), _expand_attn(a_dst1)
    eas2, ead2 = _expand_attn(a_src2), _expand_attn(a_dst2)
    eexp = jnp.repeat(jnp.eye(H, dtype=f32), C, axis=1)
    wlp = jnp.pad(Wlin, ((0, 0), (0, 128 - Wlin.shape[1])))
    blp = jnp.pad(blin.reshape(1, -1), ((0, 0), (0, 128 - blin.shape[0])))

    xp1, als1, ald1, ees1 = _tc_pre(zp, W1, eas1, ead1)
    raw1, den1 = _sc_msgpass(src, dst, als1, ald1, xp1)
    xp2, als2, ald2, ees2 = _tc_mid(raw1, den1, ees1, xp1, b1.reshape(1, F),
                                    eexp, W2, eas2, ead2)
    raw2, den2 = _sc_msgpass(src, dst, als2, ald2, xp2)
    y = _tc_fin(raw2, den2, ees2, xp2, b2.reshape(1, F), eexp, wlp, blp)
    return y[:NN, :1]
